# baseline (device time: 62155 ns/iter reference)
import jax
import jax.numpy as jnp
from jax import lax
from jax.experimental import pallas as pl
from jax.experimental.pallas import tpu as pltpu

N_DEV = 8
B = 2
SQ = 512
D_MODEL = 768
DH = 64
HQ_LOC = 8
D_LOC = HQ_LOC * DH


def kernel(x, Wq, K_ext, V_ext, Wo):
    my = lax.axis_index("i")
    wq_loc = lax.dynamic_slice_in_dim(Wq, my * D_LOC, D_LOC, axis=1)
    wo_loc = lax.dynamic_slice_in_dim(Wo, my * D_LOC, D_LOC, axis=0)
    k_t = jnp.transpose(K_ext, (0, 2, 1, 3))
    v_t = jnp.transpose(V_ext, (0, 2, 1, 3))

    def body(x_ref, wq_ref, k_ref, v_ref, wo_ref, out_ref,
             acc_ref, ctx_ref, g_ref, r0_ref, r1_ref, r2_ref,
             send_sems, recv_sems):
        p = lax.axis_index("i")
        pz = (p + 4) % 8
        py = (p // 4) * 4 + 3 - (p % 4)
        px = p + 1 - 2 * (p % 2)

        wq = wq_ref[:].astype(jnp.bfloat16)
        wo = wo_ref[:].astype(jnp.bfloat16)
        for b in range(B):
            xb = x_ref[b].astype(jnp.bfloat16)
            qb = lax.dot(xb, wq, preferred_element_type=jnp.float32)
            qb = qb.astype(jnp.bfloat16)
            for h in range(HQ_LOC):
                kh = k_ref[b, h].astype(jnp.bfloat16)
                vh = v_ref[b, h].astype(jnp.bfloat16)
                for m in range(4):
                    r1 = slice(m * DH, (m + 1) * DH)
                    r2 = slice(m * DH + 256, (m + 1) * DH + 256)
                    hc = slice(h * DH, (h + 1) * DH)
                    qg = jnp.concatenate([qb[r1, hc], qb[r2, hc]], axis=0)
                    kg = jnp.concatenate([kh[r1], kh[r2]], axis=0)
                    s = lax.dot_general(
                        qg, kg, (((1,), (1,)), ((), ())),
                        preferred_element_type=jnp.float32) * 0.125
                    s = s - jnp.max(s, axis=1, keepdims=True)
                    w = jnp.exp(s)
                    w = (w / jnp.sum(w, axis=1, keepdims=True)).astype(
                        jnp.bfloat16)
                    vg = jnp.concatenate([vh[r1], vh[r2]], axis=0)
                    cg = lax.dot(w, vg, preferred_element_type=jnp.float32
                                 ).astype(jnp.bfloat16)
                    ctx_ref[b, r1, hc] = cg[:DH]
                    ctx_ref[b, r2, hc] = cg[DH:]
            acc_ref[b] = lax.dot(ctx_ref[b], wo,
                                 preferred_element_type=jnp.float32
                                 ).astype(jnp.bfloat16)

        partner = {"z": pz, "y": py, "x": px}
        side = {"z": p // 4, "y": (p % 4) // 2, "x": p % 2}
        orders = (("z", "y", "x"), ("y", "x", "z"))
        sizes = (256, 128, 64)
        stages = (r0_ref, r1_ref, r2_ref)
        keep, send_at = [], []
        for j in range(len(orders)):
            d0, d1, d2 = orders[j]
            k0 = 256 * side[d0]
            k1 = k0 + 128 * side[d1]
            k2 = k1 + 64 * side[d2]
            keep.append((k0, k1, k2))
            send_at.append((256 * (1 - side[d0]),
                            k0 + 128 * (1 - side[d1]),
                            k1 + 64 * (1 - side[d2])))

        for s in range(3):
            rdmas = []
            for j in range(len(orders)):
                rdma = pltpu.make_async_remote_copy(
                    src_ref=acc_ref.at[:, pl.ds(send_at[j][s], sizes[s]),
                                       j * 384:(j + 1) * 384],
                    dst_ref=stages[s].at[j],
                    send_sem=send_sems.at[s * 2 + j],
                    recv_sem=recv_sems.at[s * 2 + j],
                    device_id=(partner[orders[j][s]],),
                    device_id_type=pl.DeviceIdType.MESH,
                )
                rdma.start()
                rdmas.append(rdma)
            for j in range(len(orders)):
                rdmas[j].wait()
                cur = acc_ref[:, pl.ds(keep[j][s], sizes[s]),
                              j * 384:(j + 1) * 384]
                acc_ref[:, pl.ds(keep[j][s], sizes[s]),
                        j * 384:(j + 1) * 384] = cur + stages[s][j]

        for j in range(len(orders)):
            g_ref[:, pl.ds(keep[j][2], 64), j * 384:(j + 1) * 384] = (
                acc_ref[:, pl.ds(keep[j][2], 64), j * 384:(j + 1) * 384])

        for t in range(3):
            lvl = 2 - t
            rdmas = []
            for j in range(len(orders)):
                sl = (slice(None), pl.ds(keep[j][lvl], sizes[lvl]),
                      slice(j * 384, (j + 1) * 384))
                rdma = pltpu.make_async_remote_copy(
                    src_ref=g_ref.at[sl], dst_ref=g_ref.at[sl],
                    send_sem=send_sems.at[(3 + t) * 2 + j],
                    recv_sem=recv_sems.at[(3 + t) * 2 + j],
                    device_id=(partner[orders[j][lvl]],),
                    device_id_type=pl.DeviceIdType.MESH,
                )
                rdma.start()
                rdmas.append(rdma)
            for j in range(len(orders)):
                rdmas[j].wait()

        out_ref[:, :, :] = g_ref[:, :, :].astype(jnp.float32)

    return pl.pallas_call(
        body,
        out_shape=jax.ShapeDtypeStruct((B, SQ, D_MODEL), jnp.float32),
        in_specs=[pl.BlockSpec(memory_space=pltpu.VMEM)] * 5,
        out_specs=pl.BlockSpec(memory_space=pltpu.VMEM),
        scratch_shapes=[
            pltpu.VMEM((B, SQ, D_MODEL), jnp.bfloat16),
            pltpu.VMEM((B, SQ, D_LOC), jnp.bfloat16),
            pltpu.VMEM((B, SQ, D_MODEL), jnp.bfloat16),
            pltpu.VMEM((2, B, 256, 384), jnp.bfloat16),
            pltpu.VMEM((2, B, 128, 384), jnp.bfloat16),
            pltpu.VMEM((2, B, 64, 384), jnp.bfloat16),
            pltpu.SemaphoreType.DMA((12,)),
            pltpu.SemaphoreType.DMA((12,)),
        ],
    )(x, wq_loc, k_t, v_t, wo_loc)


# device time: 48708 ns/iter; 1.2761x vs baseline; 1.2761x over previous
import jax
import jax.numpy as jnp
from jax import lax
from jax.experimental import pallas as pl
from jax.experimental.pallas import tpu as pltpu

N_DEV = 8
B = 2
SQ = 512
D_MODEL = 768
DH = 64
HQ_LOC = 8
D_LOC = HQ_LOC * DH


def kernel(x, Wq, K_ext, V_ext, Wo):
    my = lax.axis_index("i")
    wq_loc = lax.dynamic_slice_in_dim(Wq, my * D_LOC, D_LOC, axis=1)
    wo_loc = lax.dynamic_slice_in_dim(Wo, my * D_LOC, D_LOC, axis=0)
    k_t = jnp.transpose(K_ext, (0, 2, 1, 3))
    v_t = jnp.transpose(V_ext, (0, 2, 1, 3))

    def body(x_ref, wq_ref, k_ref, v_ref, wo_ref, out_ref,
             acc_ref, ctx_ref, g_ref, r0_ref, r1_ref, r2_ref,
             send_sems, recv_sems):
        p = lax.axis_index("i")
        pz = (p + 4) % 8
        py = (p // 4) * 4 + 3 - (p % 4)
        px = p + 1 - 2 * (p % 2)

        r = lax.broadcasted_iota(jnp.int32, (SQ, SQ), 0)
        c = lax.broadcasted_iota(jnp.int32, (SQ, SQ), 1)
        bias = jnp.where(((r // DH) % 4) == ((c // DH) % 4), 0.0, -30.0)

        wq = (wq_ref[:] * 0.125).astype(jnp.bfloat16)
        wo = wo_ref[:].astype(jnp.bfloat16)
        for b in range(B):
            xb = x_ref[b].astype(jnp.bfloat16)
            qb = lax.dot(xb, wq, preferred_element_type=jnp.float32)
            qb = qb.astype(jnp.bfloat16)
            for h in range(HQ_LOC):
                qh = qb[:, h * DH:(h + 1) * DH]
                kh = k_ref[b, h].astype(jnp.bfloat16)
                s = lax.dot_general(
                    qh, kh, (((1,), (1,)), ((), ())),
                    preferred_element_type=jnp.float32) + bias
                w = jnp.exp(s).astype(jnp.bfloat16)
                denom = jnp.sum(w.astype(jnp.float32), axis=1, keepdims=True)
                vh = v_ref[b, h].astype(jnp.bfloat16)
                wv = lax.dot(w, vh, preferred_element_type=jnp.float32)
                ctx_ref[b, :, h * DH:(h + 1) * DH] = (
                    wv / denom).astype(jnp.bfloat16)
            acc_ref[b] = lax.dot(ctx_ref[b], wo,
                                 preferred_element_type=jnp.float32
                                 ).astype(jnp.bfloat16)

        partner = {"z": pz, "y": py, "x": px}
        side = {"z": p // 4, "y": (p % 4) // 2, "x": p % 2}
        orders = (("z", "y", "x"), ("y", "x", "z"))
        sizes = (256, 128, 64)
        stages = (r0_ref, r1_ref, r2_ref)
        keep, send_at = [], []
        for j in range(len(orders)):
            d0, d1, d2 = orders[j]
            k0 = 256 * side[d0]
            k1 = k0 + 128 * side[d1]
            k2 = k1 + 64 * side[d2]
            keep.append((k0, k1, k2))
            send_at.append((256 * (1 - side[d0]),
                            k0 + 128 * (1 - side[d1]),
                            k1 + 64 * (1 - side[d2])))

        for s in range(3):
            rdmas = []
            for j in range(len(orders)):
                rdma = pltpu.make_async_remote_copy(
                    src_ref=acc_ref.at[:, pl.ds(send_at[j][s], sizes[s]),
                                       j * 384:(j + 1) * 384],
                    dst_ref=stages[s].at[j],
                    send_sem=send_sems.at[s * 2 + j],
                    recv_sem=recv_sems.at[s * 2 + j],
                    device_id=(partner[orders[j][s]],),
                    device_id_type=pl.DeviceIdType.MESH,
                )
                rdma.start()
                rdmas.append(rdma)
            for j in range(len(orders)):
                rdmas[j].wait()
                cur = acc_ref[:, pl.ds(keep[j][s], sizes[s]),
                              j * 384:(j + 1) * 384]
                acc_ref[:, pl.ds(keep[j][s], sizes[s]),
                        j * 384:(j + 1) * 384] = cur + stages[s][j]

        for j in range(len(orders)):
            g_ref[:, pl.ds(keep[j][2], 64), j * 384:(j + 1) * 384] = (
                acc_ref[:, pl.ds(keep[j][2], 64), j * 384:(j + 1) * 384])

        for t in range(3):
            lvl = 2 - t
            rdmas = []
            for j in range(len(orders)):
                sl = (slice(None), pl.ds(keep[j][lvl], sizes[lvl]),
                      slice(j * 384, (j + 1) * 384))
                rdma = pltpu.make_async_remote_copy(
                    src_ref=g_ref.at[sl], dst_ref=g_ref.at[sl],
                    send_sem=send_sems.at[(3 + t) * 2 + j],
                    recv_sem=recv_sems.at[(3 + t) * 2 + j],
                    device_id=(partner[orders[j][lvl]],),
                    device_id_type=pl.DeviceIdType.MESH,
                )
                rdma.start()
                rdmas.append(rdma)
            for j in range(len(orders)):
                rdmas[j].wait()

        out_ref[:, :, :] = g_ref[:, :, :].astype(jnp.float32)

    return pl.pallas_call(
        body,
        out_shape=jax.ShapeDtypeStruct((B, SQ, D_MODEL), jnp.float32),
        in_specs=[pl.BlockSpec(memory_space=pltpu.VMEM)] * 5,
        out_specs=pl.BlockSpec(memory_space=pltpu.VMEM),
        scratch_shapes=[
            pltpu.VMEM((B, SQ, D_MODEL), jnp.bfloat16),
            pltpu.VMEM((B, SQ, D_LOC), jnp.bfloat16),
            pltpu.VMEM((B, SQ, D_MODEL), jnp.bfloat16),
            pltpu.VMEM((2, B, 256, 384), jnp.bfloat16),
            pltpu.VMEM((2, B, 128, 384), jnp.bfloat16),
            pltpu.VMEM((2, B, 64, 384), jnp.bfloat16),
            pltpu.SemaphoreType.DMA((12,)),
            pltpu.SemaphoreType.DMA((12,)),
        ],
    )(x, wq_loc, k_t, v_t, wo_loc)


# device time: 44500 ns/iter; 1.3967x vs baseline; 1.0946x over previous
import jax
import jax.numpy as jnp
from jax import lax
from jax.experimental import pallas as pl
from jax.experimental.pallas import tpu as pltpu

N_DEV = 8
B = 2
SQ = 512
D_MODEL = 768
DH = 64
HQ_LOC = 8
D_LOC = HQ_LOC * DH
CHUNK = SQ // N_DEV


def kernel(x, Wq, K_ext, V_ext, Wo):
    my = lax.axis_index("i")
    wq_loc = lax.dynamic_slice_in_dim(Wq, my * D_LOC, D_LOC, axis=1)
    wo_loc = lax.dynamic_slice_in_dim(Wo, my * D_LOC, D_LOC, axis=0)
    k_t = jnp.transpose(K_ext, (0, 2, 1, 3))
    v_t = jnp.transpose(V_ext, (0, 2, 1, 3))

    def body(x_ref, wq_ref, k_ref, v_ref, wo_ref, out_ref,
             acc_ref, ctx_ref, g_ref, stage_ref,
             s1_sems, r1_sems, s2_sems, r2_sems):
        p = lax.axis_index("i")

        r = lax.broadcasted_iota(jnp.int32, (SQ, SQ), 0)
        c = lax.broadcasted_iota(jnp.int32, (SQ, SQ), 1)
        bias = jnp.where(((r // DH) % 4) == ((c // DH) % 4), 0.0, -30.0)

        wq = (wq_ref[:] * 0.125).astype(jnp.bfloat16)
        wo = wo_ref[:].astype(jnp.bfloat16)
        qb = [lax.dot(x_ref[b].astype(jnp.bfloat16), wq,
                      preferred_element_type=jnp.float32).astype(jnp.bfloat16)
              for b in range(B)]

        for half in range(2):
            rows = slice(256 * half, 256 * (half + 1))
            for b in range(B):
                for h in range(HQ_LOC):
                    hc = slice(h * DH, (h + 1) * DH)
                    kh = k_ref[b, h].astype(jnp.bfloat16)
                    s = lax.dot_general(
                        qb[b][rows, hc], kh, (((1,), (1,)), ((), ())),
                        preferred_element_type=jnp.float32) + bias[rows]
                    w = jnp.exp(s).astype(jnp.bfloat16)
                    denom = jnp.sum(w.astype(jnp.float32),
                                    axis=1, keepdims=True)
                    vh = v_ref[b, h].astype(jnp.bfloat16)
                    wv = lax.dot(w, vh, preferred_element_type=jnp.float32)
                    ctx_ref[b, rows, hc] = (wv / denom).astype(jnp.bfloat16)
                acc_ref[b, rows, :] = lax.dot(
                    ctx_ref[b, rows, :], wo,
                    preferred_element_type=jnp.float32).astype(jnp.bfloat16)
            for q in range(4 * half, 4 * half + 4):
                @pl.when(q != p)
                def _():
                    rdma = pltpu.make_async_remote_copy(
                        src_ref=acc_ref.at[:, q * CHUNK:(q + 1) * CHUNK, :],
                        dst_ref=stage_ref.at[p],
                        send_sem=s1_sems.at[q], recv_sem=r1_sems.at[p],
                        device_id=(q,), device_id_type=pl.DeviceIdType.MESH,
                    )
                    rdma.start()

        stage_ref[p] = acc_ref[:, pl.ds(p * CHUNK, CHUNK), :]
        for d in range(N_DEV):
            @pl.when(d != p)
            def _():
                pltpu.make_async_remote_copy(
                    src_ref=stage_ref.at[d], dst_ref=stage_ref.at[d],
                    send_sem=s1_sems.at[d], recv_sem=r1_sems.at[d],
                    device_id=(d,), device_id_type=pl.DeviceIdType.MESH,
                ).wait_recv()

        red = jnp.sum(stage_ref[:, :, :, :].astype(jnp.float32), axis=0)
        g_ref[:, pl.ds(p * CHUNK, CHUNK), :] = red.astype(jnp.bfloat16)

        for q in range(N_DEV):
            @pl.when(q != p)
            def _():
                pltpu.make_async_remote_copy(
                    src_ref=g_ref.at[:, pl.ds(p * CHUNK, CHUNK), :],
                    dst_ref=g_ref.at[:, pl.ds(p * CHUNK, CHUNK), :],
                    send_sem=s2_sems.at[q], recv_sem=r2_sems.at[p],
                    device_id=(q,), device_id_type=pl.DeviceIdType.MESH,
                ).start()
        for d in range(N_DEV):
            @pl.when(d != p)
            def _():
                pltpu.make_async_remote_copy(
                    src_ref=g_ref.at[:, d * CHUNK:(d + 1) * CHUNK, :],
                    dst_ref=g_ref.at[:, d * CHUNK:(d + 1) * CHUNK, :],
                    send_sem=s2_sems.at[d], recv_sem=r2_sems.at[d],
                    device_id=(d,), device_id_type=pl.DeviceIdType.MESH,
                ).wait_recv()

        out_ref[:, :, :] = g_ref[:, :, :].astype(jnp.float32)

        for q in range(N_DEV):
            @pl.when(q != p)
            def _():
                pltpu.make_async_remote_copy(
                    src_ref=acc_ref.at[:, q * CHUNK:(q + 1) * CHUNK, :],
                    dst_ref=stage_ref.at[p],
                    send_sem=s1_sems.at[q], recv_sem=r1_sems.at[p],
                    device_id=(q,), device_id_type=pl.DeviceIdType.MESH,
                ).wait_send()
                pltpu.make_async_remote_copy(
                    src_ref=g_ref.at[:, pl.ds(p * CHUNK, CHUNK), :],
                    dst_ref=g_ref.at[:, pl.ds(p * CHUNK, CHUNK), :],
                    send_sem=s2_sems.at[q], recv_sem=r2_sems.at[p],
                    device_id=(q,), device_id_type=pl.DeviceIdType.MESH,
                ).wait_send()

    return pl.pallas_call(
        body,
        out_shape=jax.ShapeDtypeStruct((B, SQ, D_MODEL), jnp.float32),
        in_specs=[pl.BlockSpec(memory_space=pltpu.VMEM)] * 5,
        out_specs=pl.BlockSpec(memory_space=pltpu.VMEM),
        scratch_shapes=[
            pltpu.VMEM((B, SQ, D_MODEL), jnp.bfloat16),
            pltpu.VMEM((B, SQ, D_LOC), jnp.bfloat16),
            pltpu.VMEM((B, SQ, D_MODEL), jnp.bfloat16),
            pltpu.VMEM((N_DEV, B, CHUNK, D_MODEL), jnp.bfloat16),
            pltpu.SemaphoreType.DMA((N_DEV,)),
            pltpu.SemaphoreType.DMA((N_DEV,)),
            pltpu.SemaphoreType.DMA((N_DEV,)),
            pltpu.SemaphoreType.DMA((N_DEV,)),
        ],
    )(x, wq_loc, k_t, v_t, wo_loc)
